# Initial kernel scaffold; baseline (speedup 1.0000x reference)
import jax
import jax.numpy as jnp
from jax.experimental import pallas as pl


def kernel(z, edge_index):
    raise NotImplementedError("write your pallas kernel here")


# speedup vs baseline: 3.9087x; 3.9087x over previous
import dataclasses
import functools

import jax
import jax.numpy as jnp
from jax import lax
from jax.experimental import pallas as pl
from jax.experimental.pallas import tpu as pltpu
from jax.experimental.pallas import tpu_sc as plsc

N_NODES = 10000
N_EDGES = 320000
D_FEAT = 128
L = 16
NUM_CORES = 2
NUM_SUBCORES = 16
NW = NUM_CORES * NUM_SUBCORES
E_PER_TILE = N_EDGES // NW
W = 80
NWIN = E_PER_TILE // W


def _sqrt_vec(x):
    bits = plsc.bitcast(x, jnp.int32)
    y = plsc.bitcast(jnp.int32(0x5F3759DF) - (bits >> 1), jnp.float32)
    half = x * 0.5
    for _ in range(3):
        y = y * (1.5 - half * y * y)
    return x * y


def kernel(z, edge_index):
    src = edge_index[0]
    dst = edge_index[1]
    mesh = plsc.VectorSubcoreMesh(core_axis_name="c", subcore_axis_name="s")
    cp = pltpu.CompilerParams()
    if "needs_layout_passes" in pltpu.CompilerParams.__dataclass_fields__:
        cp = dataclasses.replace(cp, needs_layout_passes=False)

    @functools.partial(
        pl.kernel,
        out_type=jax.ShapeDtypeStruct((N_EDGES,), jnp.float32),
        mesh=mesh,
        compiler_params=cp,
        scratch_types=[
            pltpu.VMEM((W,), jnp.int32),
            pltpu.VMEM((W,), jnp.int32),
            pltpu.VMEM((W, D_FEAT), jnp.float32),
            pltpu.VMEM((W, D_FEAT), jnp.float32),
            pltpu.VMEM((W,), jnp.float32),
            pltpu.SemaphoreType.DMA,
            pltpu.SemaphoreType.DMA,
        ],
    )
    def sc_kernel(z_hbm, src_hbm, dst_hbm, out_hbm,
                  idx_s, idx_d, rows_s, rows_d, out_v, sem_s, sem_d):
        wid = lax.axis_index("s") * NUM_CORES + lax.axis_index("c")
        tile_base = wid * E_PER_TILE

        @pl.loop(0, NWIN)
        def _window(w):
            base = pl.multiple_of(tile_base + w * W, W)
            pltpu.sync_copy(src_hbm.at[pl.ds(base, W)], idx_s)
            pltpu.sync_copy(dst_hbm.at[pl.ds(base, W)], idx_d)
            cp_s = pltpu.async_copy(z_hbm.at[idx_s], rows_s, sem_s)
            cp_d = pltpu.async_copy(z_hbm.at[idx_d], rows_d, sem_d)
            cp_s.wait()
            cp_d.wait()

            lane = lax.broadcasted_iota(jnp.int32, (L,), 0)

            @pl.loop(0, W // L)
            def _group(g):
                e0 = pl.multiple_of(g * L, L)

                def body(k, res):
                    e = e0 + k
                    acc = jnp.zeros((L,), jnp.float32)
                    for j in range(D_FEAT // L):
                        vs = rows_s[e, pl.ds(j * L, L)]
                        vd = rows_d[e, pl.ds(j * L, L)]
                        df = vs - vd
                        acc = acc + df * df
                    return jnp.where(lane == k, jnp.sum(acc), res)

                d2 = lax.fori_loop(0, L, body, jnp.zeros((L,), jnp.float32))
                out_v[pl.ds(e0, L)] = jnp.exp(-_sqrt_vec(d2))

            pltpu.sync_copy(out_v, out_hbm.at[pl.ds(base, W)])

    return sc_kernel(z, src, dst)


# speedup vs baseline: 6.2129x; 1.5895x over previous
import dataclasses
import functools

import jax
import jax.numpy as jnp
from jax import lax
from jax.experimental import pallas as pl
from jax.experimental.pallas import tpu as pltpu
from jax.experimental.pallas import tpu_sc as plsc

N_NODES = 10000
N_EDGES = 320000
D_FEAT = 128
L = 16
NUM_CORES = 2
NUM_SUBCORES = 16
NW = NUM_CORES * NUM_SUBCORES
E_PER_TILE = N_EDGES // NW
W = 80
NWIN = E_PER_TILE // W


def _sqrt_vec(x):
    bits = plsc.bitcast(x, jnp.int32)
    y = plsc.bitcast(jnp.int32(0x5F3759DF) - (bits >> 1), jnp.float32)
    half = x * 0.5
    for _ in range(3):
        y = y * (1.5 - half * y * y)
    return x * y


def kernel(z, edge_index):
    src = edge_index[0]
    dst = edge_index[1]
    mesh = plsc.VectorSubcoreMesh(core_axis_name="c", subcore_axis_name="s")
    cp = pltpu.CompilerParams()
    if "needs_layout_passes" in pltpu.CompilerParams.__dataclass_fields__:
        cp = dataclasses.replace(cp, needs_layout_passes=False)

    @functools.partial(
        pl.kernel,
        out_type=jax.ShapeDtypeStruct((N_EDGES,), jnp.float32),
        mesh=mesh,
        compiler_params=cp,
        scratch_types=[
            pltpu.VMEM((W,), jnp.int32), pltpu.VMEM((W,), jnp.int32),
            pltpu.VMEM((W,), jnp.int32), pltpu.VMEM((W,), jnp.int32),
            pltpu.VMEM((W, D_FEAT), jnp.float32),
            pltpu.VMEM((W, D_FEAT), jnp.float32),
            pltpu.VMEM((W, D_FEAT), jnp.float32),
            pltpu.VMEM((W, D_FEAT), jnp.float32),
            pltpu.VMEM((W,), jnp.float32),
            pltpu.SemaphoreType.DMA, pltpu.SemaphoreType.DMA,
            pltpu.SemaphoreType.DMA, pltpu.SemaphoreType.DMA,
        ],
    )
    def sc_kernel(z_hbm, src_hbm, dst_hbm, out_hbm,
                  idx_s0, idx_d0, idx_s1, idx_d1,
                  rows_s0, rows_d0, rows_s1, rows_d1,
                  out_v, sem_s0, sem_d0, sem_s1, sem_d1):
        wid = lax.axis_index("s") * NUM_CORES + lax.axis_index("c")
        tile_base = wid * E_PER_TILE
        bufs = [
            (idx_s0, idx_d0, rows_s0, rows_d0, sem_s0, sem_d0),
            (idx_s1, idx_d1, rows_s1, rows_d1, sem_s1, sem_d1),
        ]
        lane = lax.broadcasted_iota(jnp.int32, (L,), 0)

        def stage(w, b):
            idx_s, idx_d, rows_s, rows_d, sem_s, sem_d = bufs[b]
            base = pl.multiple_of(tile_base + w * W, W)
            pltpu.sync_copy(src_hbm.at[pl.ds(base, W)], idx_s)
            pltpu.sync_copy(dst_hbm.at[pl.ds(base, W)], idx_d)
            pltpu.async_copy(z_hbm.at[idx_s], rows_s, sem_s)
            pltpu.async_copy(z_hbm.at[idx_d], rows_d, sem_d)

        def wait(b):
            idx_s, idx_d, rows_s, rows_d, sem_s, sem_d = bufs[b]
            pltpu.make_async_copy(z_hbm.at[idx_s], rows_s, sem_s).wait()
            pltpu.make_async_copy(z_hbm.at[idx_d], rows_d, sem_d).wait()

        def compute(w, b):
            _, _, rows_s, rows_d, _, _ = bufs[b]
            base = pl.multiple_of(tile_base + w * W, W)

            @pl.loop(0, W // L)
            def _group(g):
                e0 = pl.multiple_of(g * L, L)

                def body(k, res):
                    e = e0 + k
                    acc = jnp.zeros((L,), jnp.float32)
                    for j in range(D_FEAT // L):
                        vs = rows_s[e, pl.ds(j * L, L)]
                        vd = rows_d[e, pl.ds(j * L, L)]
                        df = vs - vd
                        acc = acc + df * df
                    return jnp.where(lane == k, jnp.sum(acc), res)

                d2 = lax.fori_loop(0, L, body, jnp.zeros((L,), jnp.float32),
                                   unroll=4)
                out_v[pl.ds(e0, L)] = jnp.exp(-_sqrt_vec(d2))

            pltpu.sync_copy(out_v, out_hbm.at[pl.ds(base, W)])

        stage(0, 0)

        @pl.loop(0, (NWIN - 1) // 2)
        def _pair(p):
            w = p * 2
            stage(w + 1, 1)
            wait(0)
            compute(w, 0)
            stage(w + 2, 0)
            wait(1)
            compute(w + 1, 1)

        wait(0)
        compute(NWIN - 1, 0)

    return sc_kernel(z, src, dst)
